# baseline (speedup 1.0000x reference)
import functools

import jax
import jax.numpy as jnp
from jax import lax
from jax.experimental import pallas as pl
from jax.experimental.pallas import tpu as pltpu
from jax.experimental.pallas import tpu_sc as plsc

N = 8192
K_SC = 2048
N_TC = N - K_SC
BM = 1024
NW = 32
RPW = K_SC // NW
CHUNKS = N // 16
RU = 4


def _round_bf16(x):
    u = jax.lax.bitcast_convert_type(x, jnp.uint32)
    u = (u + jnp.uint32(0x7FFF) + ((u >> 16) & jnp.uint32(1))) & jnp.uint32(0xFFFF0000)
    return jax.lax.bitcast_convert_type(u, jnp.float32)


def _split3(x):
    mask = jnp.uint32(0xFFFF0000)

    def trunc(v):
        return jax.lax.bitcast_convert_type(
            jax.lax.bitcast_convert_type(v, jnp.uint32) & mask, jnp.float32)

    m1 = trunc(x)
    r1 = x - m1
    m2 = trunc(r1)
    r2 = r1 - m2
    m3 = trunc(r2)
    return m1, m2, m3


def _tc_body(a_ref, b_ref, colmin_ref, rowsum_ref):
    i = pl.program_id(0)

    @pl.when(i == 0)
    def _init():
        colmin_ref[...] = jnp.full((1, N), jnp.inf, dtype=jnp.float32)
        rowsum_ref[0, 0] = 0.0

    W = jax.lax.dot_general(
        a_ref[...], b_ref[...],
        dimension_numbers=(((1,), (0,)), ((), ())),
        preferred_element_type=jnp.float32,
    )

    rmin = jnp.min(W, axis=1, keepdims=True)
    rowsum_ref[0, 0] += jnp.sum(jnp.maximum(rmin, 0.0))
    colmin_ref[...] = jnp.minimum(colmin_ref[...], jnp.min(W, axis=0, keepdims=True))


def _sc_body(xt0_hbm, xt1_hbm, nt_hbm, c0_hbm, c1_hbm, nc_hbm,
             colpart_hbm, rowpart_hbm,
             xt0v, xt1v, ntv, c0v, c1v, ncv, colvec, rowvals):
    wid = lax.axis_index("s") * 2 + lax.axis_index("c")
    base = wid * RPW

    pltpu.sync_copy(xt0_hbm, xt0v)
    pltpu.sync_copy(xt1_hbm, xt1v)
    pltpu.sync_copy(nt_hbm, ntv)
    pltpu.sync_copy(c0_hbm.at[pl.ds(base * 16, RPW * 16)], c0v)
    pltpu.sync_copy(c1_hbm.at[pl.ds(base * 16, RPW * 16)], c1v)
    pltpu.sync_copy(nc_hbm.at[pl.ds(base * 16, RPW * 16)], ncv)

    inf16 = jnp.full((16,), jnp.inf, dtype=jnp.float32)

    def init_loop(k, carry):
        colvec[pl.ds(k * 16, 16)] = inf16
        return carry

    lax.fori_loop(0, CHUNKS, init_loop, 0, unroll=4)

    def row_block(rb, carry):
        r0 = rb * RU
        c0s = [c0v[pl.ds((r0 + q) * 16, 16)] for q in range(RU)]
        c1s = [c1v[pl.ds((r0 + q) * 16, 16)] for q in range(RU)]
        ncs = [ncv[pl.ds((r0 + q) * 16, 16)] for q in range(RU)]

        def chunk_loop(k, raccs):
            sl = pl.ds(k * 16, 16)
            x0 = xt0v[sl]
            x1 = xt1v[sl]
            nt16 = ntv[sl]
            cv = colvec[sl]
            out = []
            for q in range(RU):
                T = c0s[q] * x0 + c1s[q] * x1 + nt16
                out.append(jnp.minimum(raccs[q], T))
                cv = jnp.minimum(cv, T + ncs[q])
            colvec[sl] = cv
            return tuple(out)

        raccs = lax.fori_loop(0, CHUNKS, chunk_loop,
                              tuple(inf16 for _ in range(RU)), unroll=2)
        for q in range(RU):
            rowvals[pl.ds((r0 + q) * 16, 16)] = ncs[q] + raccs[q]
        return carry

    lax.fori_loop(0, RPW // RU, row_block, 0)

    pltpu.sync_copy(colvec, colpart_hbm.at[wid])
    pltpu.sync_copy(rowvals, rowpart_hbm.at[wid])


def _combine_body(colpart_ref, tccol_ref, rowpart_ref, tcrowsum_ref, out_ref):
    cmin = jnp.minimum(jnp.min(colpart_ref[...], axis=0, keepdims=True),
                       tccol_ref[...])
    colsum = jnp.sum(jnp.maximum(cmin, 0.0))
    rmins = jnp.min(rowpart_ref[...], axis=1)
    rowsum = tcrowsum_ref[0, 0] + jnp.sum(jnp.maximum(rmins, 0.0))
    out_ref[0, 0] = (rowsum + colsum) / N


def kernel(Xc, Xt):
    xc0 = Xc[:, 0]
    xc1 = Xc[:, 1]
    xt0 = Xt[:, 0]
    xt1 = Xt[:, 1]
    nc = xc0 * xc0 + xc1 * xc1
    nt = xt0 * xt0 + xt1 * xt1

    a01 = (-2.0 * Xc[:N_TC]).astype(jnp.bfloat16)
    n1, n2, n3 = _split3(nc[:N_TC])
    A = jnp.concatenate([
        a01,
        n1.astype(jnp.bfloat16).reshape(N_TC, 1),
        n2.astype(jnp.bfloat16).reshape(N_TC, 1),
        n3.astype(jnp.bfloat16).reshape(N_TC, 1),
        jnp.ones((N_TC, 3), dtype=jnp.bfloat16),
    ], axis=1)
    m1, m2, m3 = _split3(nt)
    B = jnp.concatenate([
        xt0.astype(jnp.bfloat16).reshape(1, N),
        xt1.astype(jnp.bfloat16).reshape(1, N),
        jnp.ones((3, N), dtype=jnp.bfloat16),
        m1.astype(jnp.bfloat16).reshape(1, N),
        m2.astype(jnp.bfloat16).reshape(1, N),
        m3.astype(jnp.bfloat16).reshape(1, N),
    ], axis=0)

    def splat16(v):
        return jnp.broadcast_to(v[:, None], (K_SC, 16)).reshape(K_SC * 16)

    sc_c0 = splat16(-2.0 * _round_bf16(xc0[N_TC:]))
    sc_c1 = splat16(-2.0 * _round_bf16(xc1[N_TC:]))
    sc_nc = splat16(nc[N_TC:])
    sc_xt0 = _round_bf16(xt0)
    sc_xt1 = _round_bf16(xt1)

    sc_call = functools.partial(
        pl.kernel,
        mesh=plsc.VectorSubcoreMesh(core_axis_name="c", subcore_axis_name="s"),
        out_type=[
            jax.ShapeDtypeStruct((NW, N), jnp.float32),
            jax.ShapeDtypeStruct((NW, RPW * 16), jnp.float32),
        ],
        scratch_types=[
            pltpu.VMEM((N,), jnp.float32),
            pltpu.VMEM((N,), jnp.float32),
            pltpu.VMEM((N,), jnp.float32),
            pltpu.VMEM((RPW * 16,), jnp.float32),
            pltpu.VMEM((RPW * 16,), jnp.float32),
            pltpu.VMEM((RPW * 16,), jnp.float32),
            pltpu.VMEM((N,), jnp.float32),
            pltpu.VMEM((RPW * 16,), jnp.float32),
        ],
    )
    colpart, rowpart = sc_call(_sc_body)(sc_xt0, sc_xt1, nt, sc_c0, sc_c1, sc_nc)

    tccol, tcrowsum = pl.pallas_call(
        _tc_body,
        grid=(N_TC // BM,),
        in_specs=[
            pl.BlockSpec((BM, 8), lambda i: (i, 0)),
            pl.BlockSpec((8, N), lambda i: (0, 0)),
        ],
        out_specs=[
            pl.BlockSpec((1, N), lambda i: (0, 0)),
            pl.BlockSpec((1, 1), lambda i: (0, 0), memory_space=pltpu.SMEM),
        ],
        out_shape=[
            jax.ShapeDtypeStruct((1, N), jnp.float32),
            jax.ShapeDtypeStruct((1, 1), jnp.float32),
        ],
    )(A, B)

    out = pl.pallas_call(
        _combine_body,
        out_specs=pl.BlockSpec(memory_space=pltpu.SMEM),
        out_shape=jax.ShapeDtypeStruct((1, 1), jnp.float32),
    )(colpart, tccol, rowpart.reshape(K_SC, 16), tcrowsum)
    return out[0, 0]


# speedup vs baseline: 2.7088x; 2.7088x over previous
import jax
import jax.numpy as jnp
from jax.experimental import pallas as pl
from jax.experimental.pallas import tpu as pltpu

N = 8192
BM = 1024


def _split3(x):
    mask = jnp.uint32(0xFFFF0000)

    def trunc(v):
        return jax.lax.bitcast_convert_type(
            jax.lax.bitcast_convert_type(v, jnp.uint32) & mask, jnp.float32)

    m1 = trunc(x)
    r1 = x - m1
    m2 = trunc(r1)
    r2 = r1 - m2
    m3 = trunc(r2)
    return m1, m2, m3


def _body(a_ref, b_ref, out_ref, colmin_ref, rowsum_ref):
    i = pl.program_id(0)

    @pl.when(i == 0)
    def _init():
        colmin_ref[...] = jnp.full((1, N), jnp.inf, dtype=jnp.float32)
        rowsum_ref[0] = 0.0

    W = jax.lax.dot_general(
        a_ref[...], b_ref[...],
        dimension_numbers=(((1,), (0,)), ((), ())),
        preferred_element_type=jnp.float32,
    )

    rmin = jnp.min(W, axis=1, keepdims=True)
    rowsum_ref[0] += jnp.sum(jnp.maximum(rmin, 0.0))
    colmin_ref[...] = jnp.minimum(colmin_ref[...], jnp.min(W, axis=0, keepdims=True))

    @pl.when(i == pl.num_programs(0) - 1)
    def _fin():
        out_ref[0, 0] = (rowsum_ref[0]
                         + jnp.sum(jnp.maximum(colmin_ref[...], 0.0))) / N


def kernel(Xc, Xt):
    a01 = (-2.0 * Xc).astype(jnp.bfloat16)
    nc = Xc[:, 0] * Xc[:, 0] + Xc[:, 1] * Xc[:, 1]
    n1, n2, n3 = _split3(nc)
    ones_c = jnp.ones((N, 3), dtype=jnp.bfloat16)
    A = jnp.concatenate([
        a01,
        n1.astype(jnp.bfloat16).reshape(N, 1),
        n2.astype(jnp.bfloat16).reshape(N, 1),
        n3.astype(jnp.bfloat16).reshape(N, 1),
        ones_c,
    ], axis=1)

    nt = Xt[:, 0] * Xt[:, 0] + Xt[:, 1] * Xt[:, 1]
    m1, m2, m3 = _split3(nt)
    B = jnp.concatenate([
        Xt[:, 0].astype(jnp.bfloat16).reshape(1, N),
        Xt[:, 1].astype(jnp.bfloat16).reshape(1, N),
        jnp.ones((3, N), dtype=jnp.bfloat16),
        m1.astype(jnp.bfloat16).reshape(1, N),
        m2.astype(jnp.bfloat16).reshape(1, N),
        m3.astype(jnp.bfloat16).reshape(1, N),
    ], axis=0)

    out = pl.pallas_call(
        _body,
        grid=(N // BM,),
        in_specs=[
            pl.BlockSpec((BM, 8), lambda i: (i, 0)),
            pl.BlockSpec((8, N), lambda i: (0, 0)),
        ],
        out_specs=pl.BlockSpec((1, 1), lambda i: (0, 0), memory_space=pltpu.SMEM),
        out_shape=jax.ShapeDtypeStruct((1, 1), jnp.float32),
        scratch_shapes=[
            pltpu.VMEM((1, N), jnp.float32),
            pltpu.SMEM((1,), jnp.float32),
        ],
    )(A, B)
    return out[0, 0]
